# baseline (speedup 1.0000x reference)
import jax
import jax.numpy as jnp
from jax.experimental import pallas as pl
from jax.experimental.pallas import tpu as pltpu

_TM = 32768


def _mlp_t_kernel(x_ref, w1_ref, b1t_ref, w2_ref, b2t_ref, out_ref):
    x = x_ref[...].astype(jnp.bfloat16)
    ht = jax.lax.dot_general(
        w1_ref[...], x, (((0,), (1,)), ((), ())),
        preferred_element_type=jnp.float32)
    ht = jnp.maximum(ht + b1t_ref[...], 0.0).astype(jnp.bfloat16)
    yt = jax.lax.dot_general(
        w2_ref[...], ht, (((0,), (0,)), ((), ())),
        preferred_element_type=jnp.float32)
    out_ref[...] = (yt + b2t_ref[...]).astype(out_ref.dtype)


def kernel(x, w1, b1, w2, b2):
    M, f_in = x.shape
    hidden = w1.shape[1]
    f_out = w2.shape[1]

    w1c = w1.astype(jnp.bfloat16)
    w2c = w2.astype(jnp.bfloat16)
    b1t = b1.reshape(hidden, 1)
    b2t = b2.reshape(f_out, 1)

    tm = min(_TM, M)
    grid = (pl.cdiv(M, tm),)

    yt = pl.pallas_call(
        _mlp_t_kernel,
        out_shape=jax.ShapeDtypeStruct((f_out, M), x.dtype),
        grid=grid,
        in_specs=[
            pl.BlockSpec((tm, f_in), lambda i: (i, 0)),
            pl.BlockSpec((f_in, hidden), lambda i: (0, 0)),
            pl.BlockSpec((hidden, 1), lambda i: (0, 0)),
            pl.BlockSpec((hidden, f_out), lambda i: (0, 0)),
            pl.BlockSpec((f_out, 1), lambda i: (0, 0)),
        ],
        out_specs=pl.BlockSpec((f_out, tm), lambda i: (0, i)),
        compiler_params=pltpu.CompilerParams(
            dimension_semantics=("parallel",)),
    )(x, w1c, b1t, w2c, b2t)

    return yt.T


# speedup vs baseline: 1.0018x; 1.0018x over previous
import jax
import jax.numpy as jnp
from jax.experimental import pallas as pl
from jax.experimental.pallas import tpu as pltpu

_TM = 16384


def _mlp_t2_kernel(xa_ref, xb_ref, w1_ref, b1t_ref, w2_ref, b2t_ref, out_ref):
    tm = xa_ref.shape[0]

    def half(x_ref):
        x = x_ref[...].astype(jnp.bfloat16)
        ht = jax.lax.dot_general(
            w1_ref[...], x, (((0,), (1,)), ((), ())),
            preferred_element_type=jnp.float32)
        ht = jnp.maximum(ht + b1t_ref[...], 0.0).astype(jnp.bfloat16)
        yt = jax.lax.dot_general(
            w2_ref[...], ht, (((0,), (0,)), ((), ())),
            preferred_element_type=jnp.float32)
        return yt + b2t_ref[...]

    out_ref[:, :tm] = half(xa_ref).astype(out_ref.dtype)
    out_ref[:, tm:] = half(xb_ref).astype(out_ref.dtype)


def kernel(x, w1, b1, w2, b2):
    M, f_in = x.shape
    hidden = w1.shape[1]
    f_out = w2.shape[1]

    w1c = w1.astype(jnp.bfloat16)
    w2c = w2.astype(jnp.bfloat16)
    b1t = b1.reshape(hidden, 1)
    b2t = b2.reshape(f_out, 1)

    tm = min(_TM, M // 2) or M
    grid = (pl.cdiv(M, 2 * tm),)

    yt = pl.pallas_call(
        _mlp_t2_kernel,
        out_shape=jax.ShapeDtypeStruct((f_out, M), x.dtype),
        grid=grid,
        in_specs=[
            pl.BlockSpec((tm, f_in), lambda i: (2 * i, 0)),
            pl.BlockSpec((tm, f_in), lambda i: (2 * i + 1, 0)),
            pl.BlockSpec((f_in, hidden), lambda i: (0, 0)),
            pl.BlockSpec((hidden, 1), lambda i: (0, 0)),
            pl.BlockSpec((hidden, f_out), lambda i: (0, 0)),
            pl.BlockSpec((f_out, 1), lambda i: (0, 0)),
        ],
        out_specs=pl.BlockSpec((f_out, 2 * tm), lambda i: (0, i)),
        compiler_params=pltpu.CompilerParams(
            dimension_semantics=("parallel",)),
    )(x, x, w1c, b1t, w2c, b2t)

    return yt.T


# speedup vs baseline: 1.0019x; 1.0001x over previous
import jax
import jax.numpy as jnp
from jax.experimental import pallas as pl
from jax.experimental.pallas import tpu as pltpu

_TM = 32768


def _mlp_t_kernel(x_ref, w1_ref, b1t_ref, w2_ref, b2t_ref, out_ref):
    x = x_ref[...].astype(jnp.bfloat16)
    ht = jax.lax.dot_general(
        w1_ref[...], x, (((0,), (1,)), ((), ())),
        preferred_element_type=jnp.float32)
    ht = jnp.maximum(ht + b1t_ref[...], 0.0).astype(jnp.bfloat16)
    yt = jax.lax.dot_general(
        w2_ref[...], ht, (((0,), (0,)), ((), ())),
        preferred_element_type=jnp.float32)
    out_ref[...] = (yt + b2t_ref[...]).astype(out_ref.dtype)


def kernel(x, w1, b1, w2, b2):
    M, f_in = x.shape
    hidden = w1.shape[1]
    f_out = w2.shape[1]

    w1c = w1.astype(jnp.bfloat16)
    w2c = w2.astype(jnp.bfloat16)
    b1t = b1.reshape(hidden, 1)
    b2t = b2.reshape(f_out, 1)

    tm = min(_TM, M)
    grid = (pl.cdiv(M, tm),)

    yt = pl.pallas_call(
        _mlp_t_kernel,
        out_shape=jax.ShapeDtypeStruct((f_out, M), x.dtype),
        grid=grid,
        in_specs=[
            pl.BlockSpec((tm, f_in), lambda i: (i, 0)),
            pl.BlockSpec((f_in, hidden), lambda i: (0, 0)),
            pl.BlockSpec((hidden, 1), lambda i: (0, 0)),
            pl.BlockSpec((hidden, f_out), lambda i: (0, 0)),
            pl.BlockSpec((f_out, 1), lambda i: (0, 0)),
        ],
        out_specs=pl.BlockSpec((f_out, tm), lambda i: (0, i)),
        compiler_params=pltpu.CompilerParams(
            dimension_semantics=("parallel",)),
    )(x, w1c, b1t, w2c, b2t)

    return yt.T
